# Initial kernel scaffold; baseline (speedup 1.0000x reference)
import jax
import jax.numpy as jnp
from jax.experimental import pallas as pl


def kernel(x, edge_index, W1, b1, gamma, beta, W2, b2, Wqkv, Wo, bo):
    raise NotImplementedError("write your pallas kernel here")


# speedup vs baseline: 1.3524x; 1.3524x over previous
import functools

import jax
import jax.numpy as jnp
from jax import lax
from jax.experimental import pallas as pl
from jax.experimental.pallas import tpu as pltpu
from jax.experimental.pallas import tpu_sc as plsc

N = 10000
E = 320000
D = 128
DW = 16
EPS_MSG = 1e-07
BN_EPS = 1e-05

NC = 2
NS = 16
NW = NC * NS
EDGES_PER_TILE = E // NW
CHUNK = 128
STAGE = 16
STAGES = 5
CHUNKS = STAGE * STAGES
TILE_EDGES_PAD = CHUNKS * CHUNK
AGG_ROWS = 10240
ROWS_PER_TILE = AGG_ROWS // NS

BQ = 1000
GRID = N // BQ


def _r_body(x_ref, r_ref):
    r_ref[...] = jnp.maximum(x_ref[...], 0.0) + EPS_MSG


def _build_r(x):
    return pl.pallas_call(
        _r_body,
        grid=(GRID,),
        in_specs=[pl.BlockSpec((BQ, D), lambda i: (i, 0))],
        out_specs=pl.BlockSpec((BQ, D), lambda i: (i, 0)),
        out_shape=jax.ShapeDtypeStruct((N, D), jnp.float32),
    )(x)


def _sc_body(src_hbm, dst_hbm, r_hbm, iota_hbm, agg_hbm, deg_hbm,
             src_v, dst_v, rows_v, ones_v, iot_v, agg_sh, deg_sh, sem):
    cid = lax.axis_index("c")
    sid = lax.axis_index("s")
    wid = sid * NC + cid

    def _zrow(i, _):
        for j in range(D // 16):
            rows_v[i, pl.ds(j * 16, 16)] = jnp.zeros((16,), jnp.float32)
        ones_v[i, :] = jnp.zeros((DW,), jnp.float32)
        return 0
    lax.fori_loop(0, CHUNK, _zrow, 0)

    pltpu.sync_copy(iota_hbm.at[sid], iot_v)
    for z in range(ROWS_PER_TILE // CHUNK):
        pltpu.sync_copy(rows_v, agg_sh.at[iot_v.at[z]])
        pltpu.sync_copy(ones_v, deg_sh.at[iot_v.at[z]])

    def _orow(i, _):
        ones_v[i, :] = jnp.ones((DW,), jnp.float32)
        return 0
    lax.fori_loop(0, CHUNK, _orow, 0)
    plsc.subcore_barrier()

    def _chunk(c, _):
        cidx = wid * CHUNKS + c
        pltpu.sync_copy(src_hbm.at[cidx], src_v)
        pltpu.sync_copy(dst_hbm.at[cidx], dst_v)
        pltpu.async_copy(r_hbm.at[src_v], rows_v, sem).wait()
        pltpu.sync_copy(rows_v, agg_sh.at[dst_v], add=True)
        pltpu.sync_copy(ones_v, deg_sh.at[dst_v], add=True)
        return 0
    lax.fori_loop(0, CHUNKS, _chunk, 0)
    plsc.subcore_barrier()

    for z in range(ROWS_PER_TILE // CHUNK):
        base = (sid * (ROWS_PER_TILE // CHUNK) + z) * CHUNK
        pltpu.sync_copy(agg_sh.at[iot_v.at[z]], rows_v)
        pltpu.sync_copy(rows_v, agg_hbm.at[cid, pl.ds(base, CHUNK)])
        pltpu.sync_copy(deg_sh.at[iot_v.at[z]], ones_v)
        pltpu.sync_copy(ones_v, deg_hbm.at[cid, pl.ds(base, CHUNK)])


def _sc_aggregate(src_p, dst_p, r):
    iota = jnp.arange(AGG_ROWS, dtype=jnp.int32).reshape(
        NS, ROWS_PER_TILE // CHUNK, CHUNK)
    mesh = plsc.VectorSubcoreMesh(core_axis_name="c", subcore_axis_name="s")
    fn = pl.kernel(
        _sc_body,
        out_type=[
            jax.ShapeDtypeStruct((NC, AGG_ROWS, D), jnp.float32),
            jax.ShapeDtypeStruct((NC, AGG_ROWS, DW), jnp.float32),
        ],
        mesh=mesh,
        scratch_types=[
            pltpu.VMEM((CHUNK,), jnp.int32),
            pltpu.VMEM((CHUNK,), jnp.int32),
            pltpu.VMEM((CHUNK, D), jnp.float32),
            pltpu.VMEM((CHUNK, DW), jnp.float32),
            pltpu.VMEM((ROWS_PER_TILE // CHUNK, CHUNK), jnp.int32),
            pltpu.VMEM_SHARED((AGG_ROWS, D), jnp.float32),
            pltpu.VMEM_SHARED((AGG_ROWS, DW), jnp.float32),
            pltpu.SemaphoreType.DMA,
        ],
    )
    return fn(src_p, dst_p, r, iota)


def _h1_body(x_ref, agg_ref, deg_ref, w1_ref, b1_ref, h1_ref, stats_ref):
    s = agg_ref[0] + agg_ref[1]
    deg = deg_ref[0, :, 0:1] + deg_ref[1, :, 0:1]
    h = x_ref[...] + s / jnp.maximum(deg, 1.0)
    h1 = jnp.dot(h, w1_ref[...], preferred_element_type=jnp.float32)
    h1 = h1 + b1_ref[...]
    h1_ref[...] = h1

    @pl.when(pl.program_id(0) == 0)
    def _():
        stats_ref[...] = jnp.zeros_like(stats_ref)

    colsum = jnp.sum(h1, axis=0, keepdims=True)
    colsq = jnp.sum(h1 * h1, axis=0, keepdims=True)
    stats_ref[0:1, :] = stats_ref[0:1, :] + colsum
    stats_ref[1:2, :] = stats_ref[1:2, :] + colsq


def _build_h1(x, agg, deg, W1, b1):
    return pl.pallas_call(
        _h1_body,
        grid=(GRID,),
        in_specs=[
            pl.BlockSpec((BQ, D), lambda i: (i, 0)),
            pl.BlockSpec((NC, BQ, D), lambda i: (0, i, 0)),
            pl.BlockSpec((NC, BQ, DW), lambda i: (0, i, 0)),
            pl.BlockSpec((D, 2 * D), lambda i: (0, 0)),
            pl.BlockSpec((1, 2 * D), lambda i: (0, 0)),
        ],
        out_specs=[
            pl.BlockSpec((BQ, 2 * D), lambda i: (i, 0)),
            pl.BlockSpec((8, 2 * D), lambda i: (0, 0)),
        ],
        out_shape=[
            jax.ShapeDtypeStruct((N, 2 * D), jnp.float32),
            jax.ShapeDtypeStruct((8, 2 * D), jnp.float32),
        ],
    )(x, agg, deg, W1, b1.reshape(1, -1))


def _qkv_body(h1_ref, stats_ref, gamma_ref, beta_ref, w2_ref, b2_ref,
              wqkv_ref, q_ref, k_ref, v_ref, x_ref):
    mu = stats_ref[0:1, :] / N
    var = stats_ref[1:2, :] / N - mu * mu
    scale = gamma_ref[...] * lax.rsqrt(var + BN_EPS)
    shift = beta_ref[...] - mu * scale
    hn = jnp.maximum(h1_ref[...] * scale + shift, 0.0)
    X = jnp.dot(hn, w2_ref[...], preferred_element_type=jnp.float32)
    X = X + b2_ref[...]
    x_ref[...] = X
    qkv = jnp.dot(X, wqkv_ref[...], preferred_element_type=jnp.float32)
    q_ref[...] = qkv[:, :D]
    k_ref[...] = qkv[:, D:2 * D]
    v_ref[...] = qkv[:, 2 * D:3 * D]


def _build_qkv(h1, stats, gamma, beta, W2, b2, Wqkv):
    outs = pl.pallas_call(
        _qkv_body,
        grid=(GRID,),
        in_specs=[
            pl.BlockSpec((BQ, 2 * D), lambda i: (i, 0)),
            pl.BlockSpec((8, 2 * D), lambda i: (0, 0)),
            pl.BlockSpec((1, 2 * D), lambda i: (0, 0)),
            pl.BlockSpec((1, 2 * D), lambda i: (0, 0)),
            pl.BlockSpec((2 * D, D), lambda i: (0, 0)),
            pl.BlockSpec((1, D), lambda i: (0, 0)),
            pl.BlockSpec((D, 3 * D), lambda i: (0, 0)),
        ],
        out_specs=[pl.BlockSpec((BQ, D), lambda i: (i, 0))] * 4,
        out_shape=[jax.ShapeDtypeStruct((N, D), jnp.float32)] * 4,
    )(h1, stats, gamma.reshape(1, -1), beta.reshape(1, -1), W2,
      b2.reshape(1, -1), Wqkv)
    return outs


def _sdpa_body(q_ref, k_ref, v_ref, x_ref, wo_ref, bo_ref, o_ref):
    sm_scale = 1.0 / float(D) ** 0.5
    q = q_ref[...] * sm_scale
    m = jnp.full((BQ, 1), -jnp.inf, jnp.float32)
    l = jnp.zeros((BQ, 1), jnp.float32)
    acc = jnp.zeros((BQ, D), jnp.float32)
    for kc in range(GRID):
        k = k_ref[pl.ds(kc * BQ, BQ), :]
        s = lax.dot_general(q, k, (((1,), (1,)), ((), ())),
                            preferred_element_type=jnp.float32)
        m_new = jnp.maximum(m, jnp.max(s, axis=1, keepdims=True))
        alpha = jnp.exp(m - m_new)
        p = jnp.exp(s - m_new)
        l = l * alpha + jnp.sum(p, axis=1, keepdims=True)
        v = v_ref[pl.ds(kc * BQ, BQ), :]
        acc = acc * alpha + jnp.dot(p, v, preferred_element_type=jnp.float32)
        m = m_new
    H = acc / l
    X = x_ref[...]
    out = jnp.dot(H + X, wo_ref[...], preferred_element_type=jnp.float32)
    o_ref[...] = out + bo_ref[...] + X


def _build_sdpa(Q, K, V, X, Wo, bo):
    return pl.pallas_call(
        _sdpa_body,
        grid=(GRID,),
        in_specs=[
            pl.BlockSpec((BQ, D), lambda i: (i, 0)),
            pl.BlockSpec((N, D), lambda i: (0, 0)),
            pl.BlockSpec((N, D), lambda i: (0, 0)),
            pl.BlockSpec((BQ, D), lambda i: (i, 0)),
            pl.BlockSpec((D, D), lambda i: (0, 0)),
            pl.BlockSpec((1, D), lambda i: (0, 0)),
        ],
        out_specs=pl.BlockSpec((BQ, D), lambda i: (i, 0)),
        out_shape=jax.ShapeDtypeStruct((N, D), jnp.float32),
    )(Q, K, V, X, Wo, bo.reshape(1, -1))


@jax.jit
def kernel(x, edge_index, W1, b1, gamma, beta, W2, b2, Wqkv, Wo, bo):
    src = edge_index[0]
    dst = edge_index[1]
    pad = NW * TILE_EDGES_PAD - E
    src_p = jnp.concatenate(
        [src, jnp.zeros((pad,), jnp.int32)]).reshape(NW * CHUNKS, CHUNK)
    dst_p = jnp.concatenate(
        [dst, jnp.full((pad,), N, jnp.int32)]).reshape(NW * CHUNKS, CHUNK)

    r = _build_r(x)
    aggf = jax.ops.segment_sum(r[src], dst, num_segments=AGG_ROWS)
    degf = jax.ops.segment_sum(
        jnp.ones((E, DW), jnp.float32), dst, num_segments=AGG_ROWS)
    agg = jnp.zeros((NC, AGG_ROWS, D), jnp.float32).at[0].set(aggf)
    deg = jnp.zeros((NC, AGG_ROWS, DW), jnp.float32).at[0].set(degf)
    h1, stats = _build_h1(x, agg, deg, W1, b1)
    Q, K, V, X = _build_qkv(h1, stats, gamma, beta, W2, b2, Wqkv)
    return _build_sdpa(Q, K, V, X, Wo, bo)
